# Initial kernel scaffold; baseline (speedup 1.0000x reference)
import functools

import jax
import jax.numpy as jnp
from jax import lax
from jax.experimental import pallas as pl
from jax.experimental.pallas import tpu as pltpu
from jax.experimental.pallas import tpu_sc as plsc

NC, NS, L = 2, 16, 16
NW = NC * NS

N = 16384 * 200
PER_TILE = N // NW
BLK = 6400
NBLK = PER_TILE // BLK

_GATHER_MODE = lax.GatherScatterMode.PROMISE_IN_BOUNDS


def _sc_body(x_hbm, wt_hbm, out_hbm, xbuf, obuf, wt_v):
    wid = lax.axis_index("s") * NC + lax.axis_index("c")
    tile_base = wid * PER_TILE

    pltpu.sync_copy(wt_hbm, wt_v)
    wcols = tuple(wt_v[c, :] for c in range(4))
    iota4 = lax.iota(jnp.int32, L) * 4

    def block(b, carry):
        off = tile_base + b * BLK
        pltpu.sync_copy(x_hbm.at[pl.ds(off, BLK)], xbuf)

        def group(g, c2):
            idx = xbuf[pl.ds(g * L, L)]
            pos = iota4 + g * (L * 4)
            for c, w in enumerate(wcols):
                vals = jnp.take_along_axis(w, idx, axis=0, mode=_GATHER_MODE)
                plsc.store_scatter(obuf, [pos + c], vals)
            return c2

        lax.fori_loop(0, BLK // L, group, 0, unroll=4)
        pltpu.sync_copy(obuf, out_hbm.at[pl.ds(off * 4, BLK * 4)])
        return carry

    lax.fori_loop(0, NBLK, block, 0)


@functools.partial(
    pl.kernel,
    out_type=jax.ShapeDtypeStruct((N * 4,), jnp.float32),
    mesh=plsc.VectorSubcoreMesh(core_axis_name="c", subcore_axis_name="s"),
    scratch_types=[
        pltpu.VMEM((BLK,), jnp.int32),
        pltpu.VMEM((BLK * 4,), jnp.float32),
        pltpu.VMEM((4, L), jnp.float32),
    ],
)
def _lookup(x_hbm, wt_hbm, out_hbm, xbuf, obuf, wt_v):
    _sc_body(x_hbm, wt_hbm, out_hbm, xbuf, obuf, wt_v)


def kernel(x, W):
    wt = jnp.zeros((4, L), jnp.float32).at[:, : W.shape[0]].set(W.T)
    flat = _lookup(x.reshape(-1), wt)
    return flat.reshape(x.shape[0], x.shape[1], W.shape[1])


# speedup vs baseline: 5.3752x; 5.3752x over previous
import functools

import jax
import jax.numpy as jnp
from jax import lax
from jax.experimental import pallas as pl
from jax.experimental.pallas import tpu as pltpu
from jax.experimental.pallas import tpu_sc as plsc

NC, NS, L = 2, 16, 16
NW = NC * NS

N = 16384 * 200
PER_TILE = N // NW
BLK = 6400
NBLK = PER_TILE // BLK

_GATHER_MODE = lax.GatherScatterMode.PROMISE_IN_BOUNDS


def _sc_body(x_hbm, wt_hbm, out_hbm, xbuf, obuf, wt_v):
    wid = lax.axis_index("s") * NC + lax.axis_index("c")
    tile_base = wid * PER_TILE

    pltpu.sync_copy(wt_hbm, wt_v)
    wcols = tuple(wt_v[c, :] for c in range(4))
    iota4 = lax.iota(jnp.int32, L) * 4

    def block(b, carry):
        off = tile_base + b * BLK
        pltpu.sync_copy(x_hbm.at[pl.ds(off, BLK)], xbuf)

        def group(g, c2):
            idx = xbuf[pl.ds(g * L, L)]
            pos = iota4 + g * (L * 4)
            for c, w in enumerate(wcols):
                vals = jnp.take_along_axis(w, idx, axis=0, mode=_GATHER_MODE)
                plsc.store_scatter(obuf, [pos + c], vals)
            return c2

        lax.fori_loop(0, BLK // L, group, 0, unroll=4)
        pltpu.sync_copy(obuf, out_hbm.at[pl.ds(off * 4, BLK * 4)])
        return carry

    lax.fori_loop(0, NBLK, block, 0)


@functools.partial(
    pl.kernel,
    out_type=jax.ShapeDtypeStruct((N * 4,), jnp.float32),
    mesh=plsc.VectorSubcoreMesh(core_axis_name="c", subcore_axis_name="s"),
    compiler_params=pltpu.CompilerParams(needs_layout_passes=False),
    scratch_types=[
        pltpu.VMEM((BLK,), jnp.int32),
        pltpu.VMEM((BLK * 4,), jnp.float32),
        pltpu.VMEM((4, L), jnp.float32),
    ],
)
def _lookup(x_hbm, wt_hbm, out_hbm, xbuf, obuf, wt_v):
    _sc_body(x_hbm, wt_hbm, out_hbm, xbuf, obuf, wt_v)


def kernel(x, W):
    wt = jnp.zeros((4, L), jnp.float32).at[:, : W.shape[0]].set(W.T)
    flat = _lookup(x.reshape(-1), wt)
    return flat.reshape(x.shape[0], x.shape[1], W.shape[1])


# speedup vs baseline: 50.3310x; 9.3635x over previous
import functools

import jax
import jax.numpy as jnp
from jax import lax
from jax.experimental import pallas as pl
from jax.experimental.pallas import tpu as pltpu
from jax.experimental.pallas import tpu_sc as plsc

NC, NS, L = 2, 16, 16
NW = NC * NS

NROW, NCOL, D = 16384, 200, 4
N = NROW * NCOL
OUTW = NCOL * D
PER_TILE = N // NW
ROWS_PER_TILE = NROW // NW
BLK_ROWS = 32
BLK = BLK_ROWS * NCOL
NBLK = ROWS_PER_TILE // BLK_ROWS

_GATHER_MODE = lax.GatherScatterMode.PROMISE_IN_BOUNDS


def _sc_body(x_hbm, wt_hbm, out_hbm, xbuf, obuf, wt_v):
    wid = lax.axis_index("s") * NC + lax.axis_index("c")
    tile_row0 = wid * ROWS_PER_TILE

    pltpu.sync_copy(wt_hbm, wt_v)
    wcols = tuple(wt_v[c, :] for c in range(4))
    iota4 = lax.iota(jnp.int32, L) * D

    def block(b, carry):
        row0 = tile_row0 + b * BLK_ROWS
        pltpu.sync_copy(x_hbm.at[pl.ds(row0 * NCOL, BLK)], xbuf)

        def group(g, c2):
            idx = xbuf[pl.ds(g * L, L)]
            p0 = iota4 + g * (L * D)
            row = p0 // OUTW
            col0 = p0 - row * OUTW
            for c, w in enumerate(wcols):
                vals = jnp.take_along_axis(w, idx, axis=0, mode=_GATHER_MODE)
                plsc.store_scatter(obuf, [row, col0 + c], vals)
            return c2

        lax.fori_loop(0, BLK // L, group, 0, unroll=4)
        pltpu.sync_copy(obuf, out_hbm.at[pl.ds(row0, BLK_ROWS), :])
        return carry

    lax.fori_loop(0, NBLK, block, 0)


@functools.partial(
    pl.kernel,
    out_type=jax.ShapeDtypeStruct((NROW, OUTW), jnp.float32),
    mesh=plsc.VectorSubcoreMesh(core_axis_name="c", subcore_axis_name="s"),
    compiler_params=pltpu.CompilerParams(needs_layout_passes=False),
    scratch_types=[
        pltpu.VMEM((BLK,), jnp.int32),
        pltpu.VMEM((BLK_ROWS, OUTW), jnp.float32),
        pltpu.VMEM((4, L), jnp.float32),
    ],
)
def _lookup(x_hbm, wt_hbm, out_hbm, xbuf, obuf, wt_v):
    _sc_body(x_hbm, wt_hbm, out_hbm, xbuf, obuf, wt_v)


def kernel(x, W):
    wt = jnp.zeros((4, L), jnp.float32).at[:, : W.shape[0]].set(W.T)
    out2d = _lookup(x.reshape(-1), wt)
    return out2d.reshape(NROW, NCOL, D)


# speedup vs baseline: 53.7392x; 1.0677x over previous
import functools

import jax
import jax.numpy as jnp
from jax import lax
from jax.experimental import pallas as pl
from jax.experimental.pallas import tpu as pltpu
from jax.experimental.pallas import tpu_sc as plsc

NC, NS, L = 2, 16, 16
NW = NC * NS

NROW, NCOL, D = 16384, 200, 4
OUTW = NCOL * D
ROWS_PER_TILE = NROW // NW
BLK_ROWS = 32
NBLK = ROWS_PER_TILE // BLK_ROWS

GROUP_OFFS = tuple(range(0, NCOL - L + 1, L)) + (NCOL - L,)

_GATHER_MODE = lax.GatherScatterMode.PROMISE_IN_BOUNDS


def _sc_body(x_hbm, wt_hbm, out_hbm, xbuf, obuf, wt_v):
    wid = lax.axis_index("s") * NC + lax.axis_index("c")
    tile_row0 = wid * ROWS_PER_TILE

    pltpu.sync_copy(wt_hbm, wt_v)
    wcols = tuple(wt_v[c, :] for c in range(4))
    iota4 = lax.iota(jnp.int32, L) * D
    col_bases = tuple(iota4 + o * D for o in GROUP_OFFS)

    def block(b, carry):
        row0 = tile_row0 + b * BLK_ROWS
        pltpu.sync_copy(x_hbm.at[pl.ds(row0, BLK_ROWS), :], xbuf)

        def row_iter(r, c2):
            rvec = jnp.full((L,), r, jnp.int32)
            for o, col_base in zip(GROUP_OFFS, col_bases):
                idx = xbuf[r, pl.ds(o, L)]
                for c, w in enumerate(wcols):
                    vals = jnp.take_along_axis(w, idx, axis=0, mode=_GATHER_MODE)
                    plsc.store_scatter(obuf, [rvec, col_base + c], vals)
            return c2

        lax.fori_loop(0, BLK_ROWS, row_iter, 0, unroll=2)
        pltpu.sync_copy(obuf, out_hbm.at[pl.ds(row0, BLK_ROWS), :])
        return carry

    lax.fori_loop(0, NBLK, block, 0)


@functools.partial(
    pl.kernel,
    out_type=jax.ShapeDtypeStruct((NROW, OUTW), jnp.float32),
    mesh=plsc.VectorSubcoreMesh(core_axis_name="c", subcore_axis_name="s"),
    compiler_params=pltpu.CompilerParams(needs_layout_passes=False),
    scratch_types=[
        pltpu.VMEM((BLK_ROWS, NCOL), jnp.int32),
        pltpu.VMEM((BLK_ROWS, OUTW), jnp.float32),
        pltpu.VMEM((4, L), jnp.float32),
    ],
)
def _lookup(x_hbm, wt_hbm, out_hbm, xbuf, obuf, wt_v):
    _sc_body(x_hbm, wt_hbm, out_hbm, xbuf, obuf, wt_v)


def kernel(x, W):
    wt = jnp.zeros((4, L), jnp.float32).at[:, : W.shape[0]].set(W.T)
    out2d = _lookup(x, wt)
    return out2d.reshape(NROW, NCOL, D)


# speedup vs baseline: 147.9256x; 2.7527x over previous
import functools

import jax
import jax.numpy as jnp
from jax import lax
from jax.experimental import pallas as pl
from jax.experimental.pallas import tpu as pltpu
from jax.experimental.pallas import tpu_sc as plsc

NC, NS, L = 2, 16, 16
NW = NC * NS

NROW, NCOL, D = 16384, 200, 4
JB, JS = 25, 8
IB, IL = 128, 128
IB_PER_W = IB // NW
NGRP = IL // L

_GATHER_MODE = lax.GatherScatterMode.PROMISE_IN_BOUNDS


def _sc_body(x_hbm, wt_hbm, out_hbm, xbuf, obuf, wt_v):
    wid = lax.axis_index("s") * NC + lax.axis_index("c")
    ib0 = wid * IB_PER_W

    pltpu.sync_copy(wt_hbm, wt_v)
    wcols = tuple(wt_v[c, :] for c in range(4))

    def jb_iter(jb, carry):
        pltpu.sync_copy(x_hbm.at[jb, pl.ds(ib0, IB_PER_W)], xbuf)

        for js in range(JS):
            for ib in range(IB_PER_W):
                for k in range(NGRP):
                    idx = xbuf[ib, js, pl.ds(k * L, L)]
                    for c, w in enumerate(wcols):
                        vals = jnp.take_along_axis(
                            w, idx, axis=0, mode=_GATHER_MODE)
                        obuf[js, ib, c, pl.ds(k * L, L)] = vals

        for js in range(JS):
            pltpu.sync_copy(
                obuf.at[js],
                out_hbm.at[jb * JS + js, pl.ds(ib0, IB_PER_W)],
            )
        return carry

    lax.fori_loop(0, JB, jb_iter, 0)


@functools.partial(
    pl.kernel,
    out_type=jax.ShapeDtypeStruct((NCOL, IB, D, IL), jnp.float32),
    mesh=plsc.VectorSubcoreMesh(core_axis_name="c", subcore_axis_name="s"),
    compiler_params=pltpu.CompilerParams(needs_layout_passes=False),
    scratch_types=[
        pltpu.VMEM((IB_PER_W, JS, IL), jnp.int32),
        pltpu.VMEM((JS, IB_PER_W, D, IL), jnp.float32),
        pltpu.VMEM((4, L), jnp.float32),
    ],
)
def _lookup(x_hbm, wt_hbm, out_hbm, xbuf, obuf, wt_v):
    _sc_body(x_hbm, wt_hbm, out_hbm, xbuf, obuf, wt_v)


def kernel(x, W):
    wt = jnp.zeros((4, L), jnp.float32).at[:, : W.shape[0]].set(W.T)
    xr = x.T.reshape(JB, JS, IB, IL).transpose(0, 2, 1, 3)
    out4 = _lookup(xr, wt)
    return out4.transpose(1, 3, 0, 2).reshape(NROW, NCOL, D)


# speedup vs baseline: 223.5405x; 1.5112x over previous
import functools

import jax
import jax.numpy as jnp
from jax import lax
from jax.experimental import pallas as pl
from jax.experimental.pallas import tpu as pltpu
from jax.experimental.pallas import tpu_sc as plsc

NC, NS, L = 2, 16, 16
NW = NC * NS

NROW, NCOL, D = 16384, 200, 4
JB, JS = 25, 8
IB, IL = 128, 128
IB_PER_W = IB // NW
IBQ = IB_PER_W // 2
NGRP = IL // L

_GATHER_MODE = lax.GatherScatterMode.PROMISE_IN_BOUNDS


def _sc_body(x_hbm, wt_hbm, out_hbm, xbuf, obuf, wt_v,
             insem0, insem1, outsem0, outsem1):
    insems = (insem0, insem1)
    outsems = (outsem0, outsem1)
    wid = lax.axis_index("s") * NC + lax.axis_index("c")
    ib0 = wid * IB_PER_W

    pltpu.sync_copy(wt_hbm, wt_v)
    wcols = tuple(wt_v[c, :] for c in range(4))

    def in_pair(jb, b):
        return (x_hbm.at[jb, pl.ds(ib0 + b * IBQ, IBQ)],
                xbuf.at[b], insems[b])

    def out_pair(jb, b):
        return (obuf.at[b],
                out_hbm.at[pl.ds(jb * JS, JS), pl.ds(ib0 + b * IBQ, IBQ)],
                outsems[b])

    def compute(b):
        for js in range(JS):
            for ib in range(IBQ):
                for k in range(NGRP):
                    idx = xbuf[b, ib, js, pl.ds(k * L, L)]
                    for c, w in enumerate(wcols):
                        vals = jnp.take_along_axis(
                            w, idx, axis=0, mode=_GATHER_MODE)
                        obuf[b, js, ib, c, pl.ds(k * L, L)] = vals

    pltpu.async_copy(*in_pair(0, 0))
    pltpu.async_copy(*in_pair(0, 1))

    def jb_iter(jb, carry):
        for b in range(2):
            pltpu.make_async_copy(*in_pair(jb, b)).wait()

            @pl.when(jb >= 1)
            def _wait_prev_out():
                pltpu.make_async_copy(*out_pair(jb - 1, b)).wait()

            compute(b)
            pltpu.async_copy(*out_pair(jb, b))

            @pl.when(jb + 1 < JB)
            def _prefetch_next():
                pltpu.async_copy(*in_pair(jb + 1, b))
        return carry

    lax.fori_loop(0, JB, jb_iter, 0)
    pltpu.make_async_copy(*out_pair(JB - 1, 0)).wait()
    pltpu.make_async_copy(*out_pair(JB - 1, 1)).wait()


@functools.partial(
    pl.kernel,
    out_type=jax.ShapeDtypeStruct((NCOL, IB, D, IL), jnp.float32),
    mesh=plsc.VectorSubcoreMesh(core_axis_name="c", subcore_axis_name="s"),
    compiler_params=pltpu.CompilerParams(needs_layout_passes=False),
    scratch_types=[
        pltpu.VMEM((2, IBQ, JS, IL), jnp.int32),
        pltpu.VMEM((2, JS, IBQ, D, IL), jnp.float32),
        pltpu.VMEM((4, L), jnp.float32),
        pltpu.SemaphoreType.DMA,
        pltpu.SemaphoreType.DMA,
        pltpu.SemaphoreType.DMA,
        pltpu.SemaphoreType.DMA,
    ],
)
def _lookup(x_hbm, wt_hbm, out_hbm, xbuf, obuf, wt_v,
            insem0, insem1, outsem0, outsem1):
    _sc_body(x_hbm, wt_hbm, out_hbm, xbuf, obuf, wt_v,
             insem0, insem1, outsem0, outsem1)


def kernel(x, W):
    wt = jnp.zeros((4, L), jnp.float32).at[:, : W.shape[0]].set(W.T)
    xr = x.T.reshape(JB, JS, IB, IL).transpose(0, 2, 1, 3)
    out4 = _lookup(xr, wt)
    return out4.transpose(1, 3, 0, 2).reshape(NROW, NCOL, D)


# speedup vs baseline: 224.8133x; 1.0057x over previous
import functools

import jax
import jax.numpy as jnp
from jax import lax
from jax.experimental import pallas as pl
from jax.experimental.pallas import tpu as pltpu
from jax.experimental.pallas import tpu_sc as plsc

NC, NS, L = 2, 16, 16
NW = NC * NS

NROW, NCOL, D = 16384, 200, 4
NVOC = 10
JB, JS = 25, 8
IB, IL = 128, 128
IB_PER_W = IB // NW
IBQ = IB_PER_W // 2
NGRP = IL // L

_GATHER_MODE = lax.GatherScatterMode.PROMISE_IN_BOUNDS


def _sc_body(x_hbm, wt_hbm, out_hbm, xbuf, obuf, wt_v,
             insem0, insem1, outsem0, outsem1):
    insems = (insem0, insem1)
    outsems = (outsem0, outsem1)
    wid = lax.axis_index("s") * NC + lax.axis_index("c")
    ib0 = wid * IB_PER_W

    pltpu.sync_copy(wt_hbm, wt_v)
    lane = jnp.minimum(lax.iota(jnp.int32, L), NVOC - 1)
    wcols = tuple(
        plsc.load_gather(wt_v, [jnp.full((L,), c, jnp.int32), lane])
        for c in range(4)
    )

    def in_pair(jb, b):
        return (x_hbm.at[jb, pl.ds(ib0 + b * IBQ, IBQ)],
                xbuf.at[b], insems[b])

    def out_pair(jb, b):
        return (obuf.at[b],
                out_hbm.at[pl.ds(jb * JS, JS), pl.ds(ib0 + b * IBQ, IBQ)],
                outsems[b])

    def compute(b):
        for js in range(JS):
            for ib in range(IBQ):
                for k in range(NGRP):
                    idx = xbuf[b, ib, js, pl.ds(k * L, L)]
                    for c, w in enumerate(wcols):
                        vals = jnp.take_along_axis(
                            w, idx, axis=0, mode=_GATHER_MODE)
                        obuf[b, js, ib, c, pl.ds(k * L, L)] = vals

    pltpu.async_copy(*in_pair(0, 0))
    pltpu.async_copy(*in_pair(0, 1))

    def jb_iter(jb, carry):
        for b in range(2):
            pltpu.make_async_copy(*in_pair(jb, b)).wait()

            @pl.when(jb >= 1)
            def _wait_prev_out():
                pltpu.make_async_copy(*out_pair(jb - 1, b)).wait()

            compute(b)
            pltpu.async_copy(*out_pair(jb, b))

            @pl.when(jb + 1 < JB)
            def _prefetch_next():
                pltpu.async_copy(*in_pair(jb + 1, b))
        return carry

    lax.fori_loop(0, JB, jb_iter, 0)
    pltpu.make_async_copy(*out_pair(JB - 1, 0)).wait()
    pltpu.make_async_copy(*out_pair(JB - 1, 1)).wait()


@functools.partial(
    pl.kernel,
    out_type=jax.ShapeDtypeStruct((NCOL, IB, D, IL), jnp.float32),
    mesh=plsc.VectorSubcoreMesh(core_axis_name="c", subcore_axis_name="s"),
    compiler_params=pltpu.CompilerParams(needs_layout_passes=False),
    scratch_types=[
        pltpu.VMEM((2, IBQ, JS, IL), jnp.int32),
        pltpu.VMEM((2, JS, IBQ, D, IL), jnp.float32),
        pltpu.VMEM((4, NVOC), jnp.float32),
        pltpu.SemaphoreType.DMA,
        pltpu.SemaphoreType.DMA,
        pltpu.SemaphoreType.DMA,
        pltpu.SemaphoreType.DMA,
    ],
)
def _lookup(x_hbm, wt_hbm, out_hbm, xbuf, obuf, wt_v,
            insem0, insem1, outsem0, outsem1):
    _sc_body(x_hbm, wt_hbm, out_hbm, xbuf, obuf, wt_v,
             insem0, insem1, outsem0, outsem1)


def kernel(x, W):
    wt = W.T
    xr = x.T.reshape(JB, JS, IB, IL).transpose(0, 2, 1, 3)
    out4 = _lookup(xr, wt)
    return out4.transpose(1, 3, 0, 2).reshape(NROW, NCOL, D)
